# baseline (speedup 1.0000x reference)
import functools

import jax
import jax.numpy as jnp
from jax import lax
from jax.experimental import pallas as pl
from jax.experimental.pallas import tpu as pltpu
from jax.experimental.pallas import tpu_sc as plsc

V = 100000
D = 100
B = 1024

_NC = 2
_NS = 16
_NW = _NC * _NS
_BPW = B // _NW

_DP = 128
_RT = 2000

_VT = 2048
_NV = -(-V // _VT)

_BANDR = 16

_CLAMP = 60.0


def _pad_body(t_ref, out_ref):
    out_ref[...] = jnp.concatenate(
        [t_ref[...], jnp.zeros((_RT, _DP - D), jnp.float32)], axis=1)


def _pad_table(table):
    return pl.pallas_call(
        _pad_body,
        grid=(V // _RT,),
        in_specs=[pl.BlockSpec((_RT, D), lambda i: (i, 0))],
        out_specs=pl.BlockSpec((_RT, _DP), lambda i: (i, 0)),
        out_shape=jax.ShapeDtypeStruct((V, _DP), jnp.float32),
    )(table)


def _sc_gather(table_p, idx):
    mesh = plsc.VectorSubcoreMesh(core_axis_name="c", subcore_axis_name="s")

    @functools.partial(
        pl.kernel,
        mesh=mesh,
        out_type=jax.ShapeDtypeStruct((B, _DP), jnp.float32),
        scratch_types=[
            pltpu.VMEM((_BPW,), jnp.int32),
            pltpu.VMEM((_BPW, _DP), jnp.float32),
            pltpu.SemaphoreType.DMA,
        ],
    )
    def gather_kernel(table_hbm, idx_hbm, out_hbm, idx_v, rows_v, sem):
        wid = lax.axis_index("s") * _NC + lax.axis_index("c")
        base = wid * _BPW
        pltpu.sync_copy(idx_hbm.at[pl.ds(base, _BPW)], idx_v)
        pltpu.async_copy(table_hbm.at[idx_v], rows_v, sem).wait()
        pltpu.sync_copy(rows_v, out_hbm.at[pl.ds(base, _BPW)])

    return gather_kernel(table_p, idx)


def _logits_body(emb1_ref, w_ref, b_ref, x3_ref, lse_ref, s_ref):
    j = pl.program_id(0)

    @pl.when(j == 0)
    def _init():
        s_ref[...] = jnp.zeros_like(s_ref)

    w_ext = jnp.concatenate([w_ref[...], b_ref[...]], axis=0)
    x = jnp.dot(
        emb1_ref[...].astype(jnp.bfloat16),
        w_ext.astype(jnp.bfloat16),
        preferred_element_type=jnp.float32,
    )
    x3_ref[...] = x[None]
    col = j * _VT + lax.broadcasted_iota(jnp.int32, (1, _VT), 1)
    xm = jnp.minimum(jnp.where(col < V, x, -1e30), _CLAMP)
    s_ref[...] += jnp.sum(jnp.exp(xm), axis=1, keepdims=True)

    @pl.when(j == _NV - 1)
    def _emit():
        lse_ref[...] = jnp.log(s_ref[...])


def _reband_body(x3_ref, lse_ref, out_ref):
    x = x3_ref[...]
    band = jnp.concatenate([x[j] for j in range(_NV)], axis=1)
    out_ref[...] = band[:, :V] - lse_ref[...]


def _tc_logsoftmax(emb, W, b2):
    emb1 = jnp.concatenate([emb, jnp.ones((B, 1), jnp.float32)], axis=1)
    x3, lse = pl.pallas_call(
        _logits_body,
        grid=(_NV,),
        in_specs=[
            pl.BlockSpec((B, D + 1), lambda j: (0, 0)),
            pl.BlockSpec((D, _VT), lambda j: (0, j)),
            pl.BlockSpec((1, _VT), lambda j: (0, j)),
        ],
        out_specs=[
            pl.BlockSpec((1, B, _VT), lambda j: (j, 0, 0)),
            pl.BlockSpec((B, 1), lambda j: (0, 0)),
        ],
        out_shape=[
            jax.ShapeDtypeStruct((_NV, B, _VT), jnp.float32),
            jax.ShapeDtypeStruct((B, 1), jnp.float32),
        ],
        scratch_shapes=[pltpu.VMEM((B, 1), jnp.float32)],
    )(emb1, W, b2)
    return pl.pallas_call(
        _reband_body,
        grid=(B // _BANDR,),
        in_specs=[
            pl.BlockSpec((_NV, _BANDR, _VT), lambda i: (0, i, 0)),
            pl.BlockSpec((_BANDR, 1), lambda i: (i, 0)),
        ],
        out_specs=pl.BlockSpec((_BANDR, V), lambda i: (i, 0)),
        out_shape=jax.ShapeDtypeStruct((B, V), jnp.float32),
    )(x3, lse)


def kernel(inputs, table, W, b):
    table_p = _pad_table(table)
    emb = _sc_gather(table_p, inputs.astype(jnp.int32))[:, :D]
    return _tc_logsoftmax(emb, W, b.reshape(1, V))


# speedup vs baseline: 1.2353x; 1.2353x over previous
import functools

import jax
import jax.numpy as jnp
from jax import lax
from jax.experimental import pallas as pl
from jax.experimental.pallas import tpu as pltpu
from jax.experimental.pallas import tpu_sc as plsc

V = 100000
D = 100
B = 1024

_NC = 2
_NS = 16
_NW = _NC * _NS
_BPW = B // _NW

_DP = 128
_RT = 2000

_VT = 2048
_NV = -(-V // _VT)

_CLAMP = 60.0


def _pad_body(t_ref, out_ref):
    out_ref[...] = jnp.concatenate(
        [t_ref[...], jnp.ones((_RT, 1), jnp.float32),
         jnp.zeros((_RT, _DP - D - 1), jnp.float32)], axis=1)


def _pad_table(table):
    return pl.pallas_call(
        _pad_body,
        grid=(V // _RT,),
        in_specs=[pl.BlockSpec((_RT, D), lambda i: (i, 0))],
        out_specs=pl.BlockSpec((_RT, _DP), lambda i: (i, 0)),
        out_shape=jax.ShapeDtypeStruct((V, _DP), jnp.float32),
    )(table)


def _sc_gather(table_p, idx):
    mesh = plsc.VectorSubcoreMesh(core_axis_name="c", subcore_axis_name="s")

    @functools.partial(
        pl.kernel,
        mesh=mesh,
        out_type=jax.ShapeDtypeStruct((B, _DP), jnp.float32),
        scratch_types=[
            pltpu.VMEM((_BPW,), jnp.int32),
            pltpu.VMEM((_BPW, _DP), jnp.float32),
            pltpu.SemaphoreType.DMA,
        ],
    )
    def gather_kernel(table_hbm, idx_hbm, out_hbm, idx_v, rows_v, sem):
        wid = lax.axis_index("s") * _NC + lax.axis_index("c")
        base = wid * _BPW
        pltpu.sync_copy(idx_hbm.at[pl.ds(base, _BPW)], idx_v)
        pltpu.async_copy(table_hbm.at[idx_v], rows_v, sem).wait()
        pltpu.sync_copy(rows_v, out_hbm.at[pl.ds(base, _BPW)])

    return gather_kernel(table_p, idx)


def _stats_body(emb1_ref, w_ref, b_ref, emb2_ref, s_ref):
    j = pl.program_id(0)

    @pl.when(j == 0)
    def _init():
        s_ref[...] = jnp.zeros_like(s_ref)

    w_ext = jnp.concatenate(
        [w_ref[...], b_ref[...],
         jnp.zeros((_DP - D - 1, _VT), jnp.float32)], axis=0)
    x = jnp.dot(
        emb1_ref[...].astype(jnp.bfloat16),
        w_ext.astype(jnp.bfloat16),
        preferred_element_type=jnp.float32,
    )
    col = j * _VT + lax.broadcasted_iota(jnp.int32, (1, _VT), 1)
    x = jnp.minimum(jnp.where(col < V, x, -1e30), _CLAMP)
    s_ref[...] += jnp.sum(jnp.exp(x), axis=1, keepdims=True)

    @pl.when(j == _NV - 1)
    def _emit():
        ecol = lax.broadcasted_iota(jnp.int32, (1, _DP), 1)
        emb2_ref[...] = jnp.where(
            ecol == D + 1, -jnp.log(s_ref[...]), emb1_ref[...])


def _write_body(emb2_ref, w_ref, b_ref, out_ref):
    w_ext = jnp.concatenate(
        [w_ref[...], b_ref[...], jnp.ones((1, _VT), jnp.float32),
         jnp.zeros((_DP - D - 2, _VT), jnp.float32)], axis=0)
    out_ref[...] = jnp.dot(
        emb2_ref[...].astype(jnp.bfloat16),
        w_ext.astype(jnp.bfloat16),
        preferred_element_type=jnp.float32,
    )


def _tc_logsoftmax(emb1, W, b2):
    emb2 = pl.pallas_call(
        _stats_body,
        grid=(_NV,),
        in_specs=[
            pl.BlockSpec((B, _DP), lambda j: (0, 0)),
            pl.BlockSpec((D, _VT), lambda j: (0, j)),
            pl.BlockSpec((1, _VT), lambda j: (0, j)),
        ],
        out_specs=pl.BlockSpec((B, _DP), lambda j: (0, 0)),
        out_shape=jax.ShapeDtypeStruct((B, _DP), jnp.float32),
        scratch_shapes=[pltpu.VMEM((B, 1), jnp.float32)],
    )(emb1, W, b2)
    return pl.pallas_call(
        _write_body,
        grid=(_NV,),
        in_specs=[
            pl.BlockSpec((B, _DP), lambda j: (0, 0)),
            pl.BlockSpec((D, _VT), lambda j: (0, j)),
            pl.BlockSpec((1, _VT), lambda j: (0, j)),
        ],
        out_specs=pl.BlockSpec((B, _VT), lambda j: (0, j)),
        out_shape=jax.ShapeDtypeStruct((B, V), jnp.float32),
    )(emb2, W, b2)


def kernel(inputs, table, W, b):
    table_p = _pad_table(table)
    emb1 = _sc_gather(table_p, inputs.astype(jnp.int32))
    return _tc_logsoftmax(emb1, W, b.reshape(1, V))
